# baseline (speedup 1.0000x reference)
import functools

import jax
import jax.numpy as jnp
from jax import lax
from jax.experimental import pallas as pl
from jax.experimental.pallas import tpu as pltpu
from jax.experimental.pallas import tpu_sc as plsc

NFRAG = 3200000
LANE = 128
TOT_ROWS = NFRAG // LANE
NC = 2
NS = 16
NW = NC * NS
GROUPS = TOT_ROWS // 8
GBASE = GROUPS // NW
GEXTRA = GROUPS - NW * GBASE
NSEG = 1000000
ACC_PAD = 1000448
STAGE_ELEMS = 8192
NBUF = 3
FULL_STAGES = (GBASE * 8 * LANE) // STAGE_ELEMS
TAIL_ELEMS = GBASE * 8 * LANE - FULL_STAGES * STAGE_ELEMS
TAIL_ELEMS_X = TAIL_ELEMS + 8 * LANE
WB_SLICE = ACC_PAD // NS
ZB = 8192


def _sc_body(idx_hbm, val_hbm, out_hbm, acc, idxv0, valv0, idxv1, valv1,
             idxv2, valv2, idxt8, valt8, idxt16, valt16, zb, sems):
    c = lax.axis_index("c")
    s = lax.axis_index("s")
    wid = s * NC + c
    idxb = (idxv0, idxv1, idxv2)
    valb = (valv0, valv1, valv2)
    start_elem = (wid * GBASE + jnp.minimum(wid, GEXTRA)) * 8 * LANE
    base0 = s * WB_SLICE

    def _z(i, carry):
        for u in range(8):
            zb[pl.ds(i * 128 + u * 16, 16)] = jnp.zeros((16,), jnp.float32)
        return carry

    lax.fori_loop(0, ZB // 128, _z, 0)
    zhs = []
    off = 0
    for sz in [ZB] * (WB_SLICE // ZB) + [WB_SLICE - (WB_SLICE // ZB) * ZB]:
        zhs.append(pltpu.async_copy(
            zb.at[pl.ds(0, sz)], acc.at[pl.ds(base0 + off, sz)],
            sems.at[6]))
        off += sz

    ins = {}

    def _stage_start(t):
        b = t % NBUF
        e0 = start_elem + t * STAGE_ELEMS
        ins[t] = (
            pltpu.async_copy(idx_hbm.at[pl.ds(e0, STAGE_ELEMS)], idxb[b],
                             sems.at[b]),
            pltpu.async_copy(val_hbm.at[pl.ds(e0, STAGE_ELEMS)], valb[b],
                             sems.at[NBUF + b]),
        )

    for t in range(min(NBUF, FULL_STAGES)):
        _stage_start(t)
    for h in zhs:
        h.wait()
    plsc.subcore_barrier()

    scs = {}
    for t in range(FULL_STAGES):
        b = t % NBUF
        for h in ins[t]:
            h.wait()
        scs[t] = pltpu.async_copy(valb[b], acc.at[idxb[b]],
                                  sems.at[7 + b], add=True)
        if t + 1 >= NBUF and t + 1 < FULL_STAGES:
            if t - 2 >= 0:
                scs[t - 2].wait()
            _stage_start(t + 1)
    for t in range(max(0, FULL_STAGES - NBUF), FULL_STAGES):
        scs[t].wait()

    tail_elem = start_elem + FULL_STAGES * STAGE_ELEMS

    def _do_tail(ib, vb):
        pltpu.sync_copy(idx_hbm.at[pl.ds(tail_elem, ib.shape[0])], ib)
        pltpu.sync_copy(val_hbm.at[pl.ds(tail_elem, vb.shape[0])], vb)
        pltpu.sync_copy(vb, acc.at[ib], add=True)

    @pl.when(wid < GEXTRA)
    def _():
        _do_tail(idxt16, valt16)

    @pl.when(wid >= GEXTRA)
    def _():
        _do_tail(idxt8, valt8)

    plsc.subcore_barrier()

    last = NSEG - (NS - 1) * WB_SLICE

    def _wb(total):
        nfull = total // STAGE_ELEMS
        sizes = [STAGE_ELEMS] * nfull
        if total - nfull * STAGE_ELEMS:
            sizes.append(total - nfull * STAGE_ELEMS)
        offs = [sum(sizes[:k]) for k in range(len(sizes))]
        inh = [None, None]
        outh = [None, None]

        def _in(k):
            b = k % 2
            if outh[b] is not None:
                outh[b].wait()
            inh[b] = pltpu.async_copy(
                acc.at[pl.ds(base0 + offs[k], sizes[k])],
                valb[b].at[pl.ds(0, sizes[k])], sems.at[b])

        _in(0)
        for k, sz in enumerate(sizes):
            b = k % 2
            if k + 1 < len(sizes):
                _in(k + 1)
            inh[b].wait()
            outh[b] = pltpu.async_copy(
                valb[b].at[pl.ds(0, sz)],
                out_hbm.at[pl.ds(c * NSEG + base0 + offs[k], sz)],
                sems.at[7 + b])
        for h in outh:
            if h is not None:
                h.wait()

    @pl.when(s < NS - 1)
    def _():
        _wb(WB_SLICE)

    @pl.when(s == NS - 1)
    def _():
        _wb(last)


@functools.partial(
    pl.kernel,
    out_type=jax.ShapeDtypeStruct((NC * NSEG,), jnp.float32),
    mesh=plsc.VectorSubcoreMesh(core_axis_name="c", subcore_axis_name="s",
                                num_cores=NC),
    scratch_types=[
        pltpu.VMEM_SHARED((ACC_PAD,), jnp.float32),
        pltpu.VMEM((STAGE_ELEMS,), jnp.int32),
        pltpu.VMEM((STAGE_ELEMS,), jnp.float32),
        pltpu.VMEM((STAGE_ELEMS,), jnp.int32),
        pltpu.VMEM((STAGE_ELEMS,), jnp.float32),
        pltpu.VMEM((STAGE_ELEMS,), jnp.int32),
        pltpu.VMEM((STAGE_ELEMS,), jnp.float32),
        pltpu.VMEM((TAIL_ELEMS,), jnp.int32),
        pltpu.VMEM((TAIL_ELEMS,), jnp.float32),
        pltpu.VMEM((TAIL_ELEMS_X,), jnp.int32),
        pltpu.VMEM((TAIL_ELEMS_X,), jnp.float32),
        pltpu.VMEM((ZB,), jnp.float32),
        pltpu.SemaphoreType.DMA((10,)),
    ],
)
def _sc_segment_sum(idx_hbm, val_hbm, out_hbm, acc, idxv0, valv0,
                    idxv1, valv1, idxv2, valv2, idxt8, valt8,
                    idxt16, valt16, zb, sems):
    _sc_body(idx_hbm, val_hbm, out_hbm, acc, idxv0, valv0, idxv1, valv1,
             idxv2, valv2, idxt8, valt8, idxt16, valt16, zb, sems)


def _combine_body(p_ref, o_ref):
    o_ref[...] = p_ref[0] + p_ref[1]


def kernel(likelihood, local_cellxgene_ix, n_cells, n_genes):
    idx1d = local_cellxgene_ix.astype(jnp.int32)
    part = _sc_segment_sum(idx1d, likelihood)
    part3 = part.reshape(NC, 1000, 1000)
    out = pl.pallas_call(
        _combine_body,
        grid=(5,),
        in_specs=[pl.BlockSpec((NC, 200, 1000), lambda i: (0, i, 0))],
        out_specs=pl.BlockSpec((200, 1000), lambda i: (i, 0)),
        out_shape=jax.ShapeDtypeStruct((1000, 1000), jnp.float32),
    )(part3)
    return out


# speedup vs baseline: 1.0168x; 1.0168x over previous
import functools

import jax
import jax.numpy as jnp
from jax import lax
from jax.experimental import pallas as pl
from jax.experimental.pallas import tpu as pltpu
from jax.experimental.pallas import tpu_sc as plsc

NFRAG = 3200000
LANE = 128
TOT_ROWS = NFRAG // LANE
NC = 2
NS = 16
NW = NC * NS
GROUPS = TOT_ROWS // 8
GBASE = GROUPS // NW
GEXTRA = GROUPS - NW * GBASE
NSEG = 1000000
ACC_PAD = 1000448
STAGE_ELEMS = 8192
NBUF = 3
FULL_STAGES = (GBASE * 8 * LANE) // STAGE_ELEMS
TAIL_ELEMS = GBASE * 8 * LANE - FULL_STAGES * STAGE_ELEMS
TAIL_ELEMS_X = TAIL_ELEMS + 8 * LANE
WB_SLICE = ACC_PAD // NS
ZB = 8192


def _sc_body(idx_hbm, val_hbm, out_hbm, acc, idxv0, valv0, idxv1, valv1,
             idxv2, valv2, idxt8, valt8, idxt16, valt16, zb, sems):
    c = lax.axis_index("c")
    s = lax.axis_index("s")
    wid = s * NC + c
    idxb = (idxv0, idxv1, idxv2)
    valb = (valv0, valv1, valv2)
    start_elem = (wid * GBASE + jnp.minimum(wid, GEXTRA)) * 8 * LANE
    base0 = s * WB_SLICE

    def _z(i, carry):
        for u in range(8):
            zb[pl.ds(i * 128 + u * 16, 16)] = jnp.zeros((16,), jnp.float32)
        return carry

    lax.fori_loop(0, ZB // 128, _z, 0)
    zhs = []
    off = 0
    for sz in [ZB] * (WB_SLICE // ZB) + [WB_SLICE - (WB_SLICE // ZB) * ZB]:
        zhs.append(pltpu.async_copy(
            zb.at[pl.ds(0, sz)], acc.at[pl.ds(base0 + off, sz)],
            sems.at[6]))
        off += sz

    ins = {}

    def _stage_start(t):
        b = t % NBUF
        e0 = start_elem + t * STAGE_ELEMS
        ins[t] = (
            pltpu.async_copy(idx_hbm.at[pl.ds(e0, STAGE_ELEMS)], idxb[b],
                             sems.at[b]),
            pltpu.async_copy(val_hbm.at[pl.ds(e0, STAGE_ELEMS)], valb[b],
                             sems.at[NBUF + b]),
        )

    for t in range(min(NBUF, FULL_STAGES)):
        _stage_start(t)
    for h in zhs:
        h.wait()
    plsc.subcore_barrier()

    scs = {}
    for t in range(FULL_STAGES):
        b = t % NBUF
        for h in ins[t]:
            h.wait()
        scs[t] = pltpu.async_copy(valb[b], acc.at[idxb[b]],
                                  sems.at[7 + b], add=True)
        if t + 1 >= NBUF and t + 1 < FULL_STAGES:
            if t - 2 >= 0:
                scs[t - 2].wait()
            _stage_start(t + 1)
    for t in range(max(0, FULL_STAGES - NBUF), FULL_STAGES):
        scs[t].wait()

    tail_elem = start_elem + FULL_STAGES * STAGE_ELEMS

    def _do_tail(ib, vb):
        pltpu.sync_copy(idx_hbm.at[pl.ds(tail_elem, ib.shape[0])], ib)
        pltpu.sync_copy(val_hbm.at[pl.ds(tail_elem, vb.shape[0])], vb)
        pltpu.sync_copy(vb, acc.at[ib], add=True)

    @pl.when(wid < GEXTRA)
    def _():
        _do_tail(idxt16, valt16)

    @pl.when(wid >= GEXTRA)
    def _():
        _do_tail(idxt8, valt8)

    plsc.subcore_barrier()

    last = NSEG - (NS - 1) * WB_SLICE

    def _wb(total):
        nfull = total // STAGE_ELEMS
        sizes = [STAGE_ELEMS] * nfull
        if total - nfull * STAGE_ELEMS:
            sizes.append(total - nfull * STAGE_ELEMS)
        offs = [sum(sizes[:k]) for k in range(len(sizes))]
        inh = [None, None]
        outh = [None, None]

        def _in(k):
            b = k % 2
            if outh[b] is not None:
                outh[b].wait()
            inh[b] = pltpu.async_copy(
                acc.at[pl.ds(base0 + offs[k], sizes[k])],
                valb[b].at[pl.ds(0, sizes[k])], sems.at[b])

        _in(0)
        for k, sz in enumerate(sizes):
            b = k % 2
            if k + 1 < len(sizes):
                _in(k + 1)
            inh[b].wait()
            outh[b] = pltpu.async_copy(
                valb[b].at[pl.ds(0, sz)],
                out_hbm.at[pl.ds(c * NSEG + base0 + offs[k], sz)],
                sems.at[7 + b])
        for h in outh:
            if h is not None:
                h.wait()

    @pl.when(s < NS - 1)
    def _():
        _wb(WB_SLICE)

    @pl.when(s == NS - 1)
    def _():
        _wb(last)


@functools.partial(
    pl.kernel,
    out_type=jax.ShapeDtypeStruct((NC * NSEG,), jnp.float32),
    mesh=plsc.VectorSubcoreMesh(core_axis_name="c", subcore_axis_name="s",
                                num_cores=NC),
    scratch_types=[
        pltpu.VMEM_SHARED((ACC_PAD,), jnp.float32),
        pltpu.VMEM((STAGE_ELEMS,), jnp.int32),
        pltpu.VMEM((STAGE_ELEMS,), jnp.float32),
        pltpu.VMEM((STAGE_ELEMS,), jnp.int32),
        pltpu.VMEM((STAGE_ELEMS,), jnp.float32),
        pltpu.VMEM((STAGE_ELEMS,), jnp.int32),
        pltpu.VMEM((STAGE_ELEMS,), jnp.float32),
        pltpu.VMEM((TAIL_ELEMS,), jnp.int32),
        pltpu.VMEM((TAIL_ELEMS,), jnp.float32),
        pltpu.VMEM((TAIL_ELEMS_X,), jnp.int32),
        pltpu.VMEM((TAIL_ELEMS_X,), jnp.float32),
        pltpu.VMEM((ZB,), jnp.float32),
        pltpu.SemaphoreType.DMA((10,)),
    ],
)
def _sc_segment_sum(idx_hbm, val_hbm, out_hbm, acc, idxv0, valv0,
                    idxv1, valv1, idxv2, valv2, idxt8, valt8,
                    idxt16, valt16, zb, sems):
    _sc_body(idx_hbm, val_hbm, out_hbm, acc, idxv0, valv0, idxv1, valv1,
             idxv2, valv2, idxt8, valt8, idxt16, valt16, zb, sems)


def _combine_body(p_ref, o_ref):
    o_ref[...] = p_ref[0] + p_ref[1]


def kernel(likelihood, local_cellxgene_ix, n_cells, n_genes):
    idx1d = local_cellxgene_ix.astype(jnp.int32)
    part = _sc_segment_sum(idx1d, likelihood)
    part3 = part.reshape(NC, 1000, 1000)
    out = pl.pallas_call(
        _combine_body,
        out_shape=jax.ShapeDtypeStruct((1000, 1000), jnp.float32),
    )(part3)
    return out
